# Initial kernel scaffold; baseline (speedup 1.0000x reference)
import jax
import jax.numpy as jnp
from jax.experimental import pallas as pl


def kernel(lonlat, slepian_proj, norm_slep_w, norm_slep_b, norm_sh_w, norm_sh_b, cache_mem):
    raise NotImplementedError("write your pallas kernel here")


# speedup vs baseline: 9.6436x; 9.6436x over previous
import functools
import math

import jax
import jax.numpy as jnp
from jax import lax
from jax.experimental import pallas as pl
from jax.experimental.pallas import tpu as pltpu
from jax.experimental.pallas import tpu_sc as plsc

L_SLEP = 20
L_SH = 10
CACHE_SIZE = 200000
SLEP_DIM = 100
SH_DIM = L_SH * L_SH
EMBED_DIM = SLEP_DIM + SH_DIM
EMBED_PAD = 256
BATCH = 16384

ENC_BLK = 512
WIN_I = 2048
WIN_J = 512


def _feat_index(l, m):
    return l * l + (m + l)


def _enc_kernel(lonlatT_ref, proj_ref, wslep_ref, bslep_ref, wsh_ref, bsh_ref,
                out_ref, feat_ref):
    lon = jnp.deg2rad(lonlatT_ref[0:1, :])
    lat = jnp.deg2rad(lonlatT_ref[1:2, :])
    x = jnp.sin(lat)
    somx2 = jnp.sqrt(jnp.clip(1.0 - x * x, 0.0, 1.0))
    cos1 = jnp.cos(lon)
    sin1 = jnp.sin(lon)

    sqrt2 = math.sqrt(2.0)
    pmm = jnp.ones_like(x)
    cm, sm = jnp.ones_like(x), jnp.zeros_like(x)
    cm1, sm1 = None, None
    for m in range(L_SLEP):
        if m > 0:
            pmm = -(2 * m - 1) * somx2 * pmm
            if m == 1:
                cm1, sm1 = cm, sm
                cm, sm = cos1, sin1
            else:
                c_new = 2.0 * cos1 * cm - cm1
                s_new = 2.0 * cos1 * sm - sm1
                cm1, sm1 = cm, sm
                cm, sm = c_new, s_new
        pl2, pl1 = None, pmm
        for l in range(m, L_SLEP):
            if l == m:
                p = pmm
            elif l == m + 1:
                p = x * (2 * m + 1) * pmm
            else:
                p = ((2 * l - 1) * x * pl1 - (l + m - 1) * pl2) / (l - m)
            if l > m:
                pl2, pl1 = pl1, p
            n = math.sqrt((2 * l + 1) / (4 * math.pi)
                          * math.factorial(l - m) / math.factorial(l + m))
            base = n * p
            if m == 0:
                feat_ref[_feat_index(l, 0):_feat_index(l, 0) + 1, :] = base
            else:
                feat_ref[_feat_index(l, m):_feat_index(l, m) + 1, :] = sqrt2 * base * cm
                feat_ref[_feat_index(l, -m):_feat_index(l, -m) + 1, :] = sqrt2 * base * sm
    featT = feat_ref[...]
    slepT = lax.dot_general(proj_ref[...], featT,
                            (((0,), (0,)), ((), ())),
                            preferred_element_type=jnp.float32,
                            precision=lax.Precision.HIGHEST)
    shT = featT[0:SH_DIM, :]

    def _ln_T(v, w_col, b_col):
        mu = jnp.mean(v, axis=0, keepdims=True)
        var = jnp.mean((v - mu) * (v - mu), axis=0, keepdims=True)
        return (v - mu) / jnp.sqrt(var + 1e-5) * w_col + b_col

    slepT = _ln_T(slepT, wslep_ref[...], bslep_ref[...])
    shT = _ln_T(shT, wsh_ref[...], bsh_ref[...])
    encT = jnp.concatenate(
        [slepT, shT, jnp.zeros((EMBED_PAD - EMBED_DIM,) + slepT.shape[1:],
                               dtype=jnp.float32)], axis=0)

    r = out_ref.shape[0]
    eye = (lax.broadcasted_iota(jnp.int32, (r, r), 0)
           == lax.broadcasted_iota(jnp.int32, (r, r), 1)).astype(jnp.float32)
    out_ref[...] = lax.dot_general(eye, encT, (((1,), (1,)), ((), ())),
                                   preferred_element_type=jnp.float32,
                                   precision=lax.Precision.HIGHEST)


def _hash_rows(q0, q1):
    h = q0 * jnp.int32(73856093) ^ q1 * jnp.int32(19349663)
    return jnp.mod(h, jnp.int32(CACHE_SIZE))


def _winner_kernel(lonlat_ref, lonlatT_blk_ref, out_ref):
    qT = jnp.round(lonlatT_blk_ref[...] * 1e4).astype(jnp.int32)
    h_i = _hash_rows(qT[0:1, :], qT[1:2, :])

    w = jnp.full((1, WIN_I), -1, dtype=jnp.int32)
    for j0 in range(0, BATCH, WIN_J):
        qj = jnp.round(lonlat_ref[j0:j0 + WIN_J, :] * 1e4).astype(jnp.int32)
        h_j = _hash_rows(qj[:, 0:1], qj[:, 1:2])
        eq = h_j == h_i
        jidx = lax.broadcasted_iota(jnp.int32, (WIN_J, WIN_I), 0) + j0
        cand = jnp.where(eq, jidx, -1)
        w = jnp.maximum(w, jnp.max(cand, axis=0, keepdims=True))
    out_ref[0, 0:1, :] = w


def _gather_build(nc, ns):
    nw = nc * ns
    b_per_w = BATCH // nw
    n_chunks = b_per_w // 128
    mesh = plsc.VectorSubcoreMesh(core_axis_name="c", subcore_axis_name="s")

    @functools.partial(
        pl.kernel, mesh=mesh,
        out_type=jax.ShapeDtypeStruct((BATCH, EMBED_PAD), jnp.float32),
        scratch_types=[
            pltpu.VMEM((n_chunks, 128), jnp.int32),
            pltpu.VMEM((2, 128, EMBED_PAD), jnp.float32),
            pltpu.SemaphoreType.DMA,
            pltpu.SemaphoreType.DMA,
        ],
    )
    def _gather(enc_hbm, widx_hbm, out_hbm, idx_v, rows_v, gsem, osem):
        wid = lax.axis_index("s") * nc + lax.axis_index("c")
        base = wid * b_per_w
        pltpu.sync_copy(widx_hbm.at[pl.ds(wid * n_chunks, n_chunks)], idx_v)
        cps = [None, None]
        ops = [None, None]
        cps[0] = pltpu.async_copy(enc_hbm.at[idx_v.at[0]], rows_v.at[0], gsem)
        for c in range(n_chunks):
            b = c % 2
            nb = (c + 1) % 2
            if c + 1 < n_chunks:
                if ops[nb] is not None:
                    ops[nb].wait()
                    ops[nb] = None
                cps[nb] = pltpu.async_copy(
                    enc_hbm.at[idx_v.at[c + 1]], rows_v.at[nb], gsem)
            cps[b].wait()
            ops[b] = pltpu.async_copy(
                rows_v.at[b], out_hbm.at[pl.ds(base + c * 128, 128)], osem)
        for op in ops:
            if op is not None:
                op.wait()

    return _gather


def kernel(lonlat, slepian_proj, norm_slep_w, norm_slep_b, norm_sh_w, norm_sh_b, cache_mem):
    del cache_mem
    lonlatT = lonlat.T
    w_slep = norm_slep_w.reshape(SLEP_DIM, 1)
    b_slep = norm_slep_b.reshape(SLEP_DIM, 1)
    w_sh = norm_sh_w.reshape(SH_DIM, 1)
    b_sh = norm_sh_b.reshape(SH_DIM, 1)

    enc = pl.pallas_call(
        _enc_kernel,
        grid=(BATCH // ENC_BLK,),
        in_specs=[
            pl.BlockSpec((2, ENC_BLK), lambda i: (0, i)),
            pl.BlockSpec((L_SLEP * L_SLEP, SLEP_DIM), lambda i: (0, 0)),
            pl.BlockSpec((SLEP_DIM, 1), lambda i: (0, 0)),
            pl.BlockSpec((SLEP_DIM, 1), lambda i: (0, 0)),
            pl.BlockSpec((SH_DIM, 1), lambda i: (0, 0)),
            pl.BlockSpec((SH_DIM, 1), lambda i: (0, 0)),
        ],
        out_specs=pl.BlockSpec((ENC_BLK, EMBED_PAD), lambda i: (i, 0)),
        out_shape=jax.ShapeDtypeStruct((BATCH, EMBED_PAD), jnp.float32),
        scratch_shapes=[pltpu.VMEM((L_SLEP * L_SLEP, ENC_BLK), jnp.float32)],
    )(lonlatT, slepian_proj, w_slep, b_slep, w_sh, b_sh)

    n_i = BATCH // WIN_I
    winner = pl.pallas_call(
        _winner_kernel,
        grid=(n_i,),
        in_specs=[
            pl.BlockSpec((BATCH, 2), lambda i: (0, 0)),
            pl.BlockSpec((2, WIN_I), lambda i: (0, i)),
        ],
        out_specs=pl.BlockSpec((1, 1, WIN_I), lambda i: (i, 0, 0)),
        out_shape=jax.ShapeDtypeStruct((n_i, 1, WIN_I), jnp.int32),
    )(lonlat, lonlatT)

    widx = winner.reshape(BATCH // 128, 128)
    info = plsc.get_sparse_core_info()
    gather = _gather_build(info.num_cores, info.num_subcores)
    return gather(enc, widx)[:, :EMBED_DIM]


# speedup vs baseline: 25.8281x; 2.6783x over previous
import functools
import math

import jax
import jax.numpy as jnp
from jax import lax
from jax.experimental import pallas as pl
from jax.experimental.pallas import tpu as pltpu
from jax.experimental.pallas import tpu_sc as plsc

L_SLEP = 20
L_SH = 10
CACHE_SIZE = 200000
SLEP_DIM = 100
SH_DIM = L_SH * L_SH
EMBED_DIM = SLEP_DIM + SH_DIM
EMBED_PAD = 256
BATCH = 16384

ENC_BLK = 512
WIN_I = 2048
WIN_J = 512


def _feat_index(l, m):
    return l * l + (m + l)


def _enc_kernel(lonlatT_ref, proj_ref, wslep_ref, bslep_ref, wsh_ref, bsh_ref,
                out_ref, feat_ref):
    lon = jnp.deg2rad(lonlatT_ref[0:1, :])
    lat = jnp.deg2rad(lonlatT_ref[1:2, :])
    x = jnp.sin(lat)
    somx2 = jnp.sqrt(jnp.clip(1.0 - x * x, 0.0, 1.0))
    cos1 = jnp.cos(lon)
    sin1 = jnp.sin(lon)

    sqrt2 = math.sqrt(2.0)
    pmm = jnp.ones_like(x)
    cm, sm = jnp.ones_like(x), jnp.zeros_like(x)
    cm1, sm1 = None, None
    for m in range(L_SLEP):
        if m > 0:
            pmm = -(2 * m - 1) * somx2 * pmm
            if m == 1:
                cm1, sm1 = cm, sm
                cm, sm = cos1, sin1
            else:
                c_new = 2.0 * cos1 * cm - cm1
                s_new = 2.0 * cos1 * sm - sm1
                cm1, sm1 = cm, sm
                cm, sm = c_new, s_new
        pl2, pl1 = None, pmm
        for l in range(m, L_SLEP):
            if l == m:
                p = pmm
            elif l == m + 1:
                p = x * (2 * m + 1) * pmm
            else:
                p = ((2 * l - 1) * x * pl1 - (l + m - 1) * pl2) / (l - m)
            if l > m:
                pl2, pl1 = pl1, p
            n = math.sqrt((2 * l + 1) / (4 * math.pi)
                          * math.factorial(l - m) / math.factorial(l + m))
            base = n * p
            if m == 0:
                feat_ref[_feat_index(l, 0):_feat_index(l, 0) + 1, :] = base
            else:
                feat_ref[_feat_index(l, m):_feat_index(l, m) + 1, :] = sqrt2 * base * cm
                feat_ref[_feat_index(l, -m):_feat_index(l, -m) + 1, :] = sqrt2 * base * sm
    featT = feat_ref[...]
    slepT = lax.dot_general(proj_ref[...], featT,
                            (((0,), (0,)), ((), ())),
                            preferred_element_type=jnp.float32,
                            precision=lax.Precision.HIGHEST)
    shT = featT[0:SH_DIM, :]

    def _ln_T(v, w_col, b_col):
        mu = jnp.mean(v, axis=0, keepdims=True)
        var = jnp.mean((v - mu) * (v - mu), axis=0, keepdims=True)
        return (v - mu) / jnp.sqrt(var + 1e-5) * w_col + b_col

    slepT = _ln_T(slepT, wslep_ref[...], bslep_ref[...])
    shT = _ln_T(shT, wsh_ref[...], bsh_ref[...])
    encT = jnp.concatenate(
        [slepT, shT, jnp.zeros((EMBED_PAD - EMBED_DIM,) + slepT.shape[1:],
                               dtype=jnp.float32)], axis=0)

    r = out_ref.shape[0]
    eye = (lax.broadcasted_iota(jnp.int32, (r, r), 0)
           == lax.broadcasted_iota(jnp.int32, (r, r), 1)).astype(jnp.float32)
    out_ref[...] = lax.dot_general(eye, encT, (((1,), (1,)), ((), ())),
                                   preferred_element_type=jnp.float32,
                                   precision=lax.Precision.HIGHEST)


def _hash_rows(q0, q1):
    h = q0 * jnp.int32(73856093) ^ q1 * jnp.int32(19349663)
    return jnp.mod(h, jnp.int32(CACHE_SIZE))


def _hash_kernel(lonlatT_ref, out_ref):
    qT = jnp.round(lonlatT_ref[...] * 1e4).astype(jnp.int32)
    out_ref[...] = _hash_rows(qT[0:1, :], qT[1:2, :])


TBL_PAD = 200192
DUMP = 200032
ROWS_PER_TILE = BATCH // 16
N_ROW_CHUNKS = ROWS_PER_TILE // 128
ZCHUNK = 3 * TBL_PAD // 16

_PASSES = (
    (11, 10, 0xF, 4, 10),
    (6, 5, 0x1F, 5, 5),
    (2, 0, 0x1F, 5, 0),
)


def _pow2_of_group(g, spacing, nbits):
    v = jnp.full((16,), 2.0 ** -126, jnp.float32)
    for b in range(nbits):
        f = jnp.float32(2.0 ** (spacing * (1 << b)))
        v = jnp.where((g & (1 << b)) != 0, v * f, v)
    return v


def _top_group(s, spacing, nbits):
    cur = jnp.full((16,), 2.0 ** -126, jnp.float32)
    g = jnp.zeros((16,), jnp.int32)
    for b in range(nbits - 1, -1, -1):
        trial = cur * jnp.float32(2.0 ** (spacing * (1 << b)))
        take = s >= trial
        cur = jnp.where(take, trial, cur)
        g = g | jnp.where(take, jnp.int32(1 << b), jnp.int32(0))
    return g


def _winner_sc_build(nc, ns):
    mesh = plsc.VectorSubcoreMesh(core_axis_name="c", subcore_axis_name="s")

    @functools.partial(
        pl.kernel, mesh=mesh,
        out_type=jax.ShapeDtypeStruct((BATCH // 128, 128), jnp.int32),
        scratch_types=[
            pltpu.VMEM((N_ROW_CHUNKS, 128), jnp.int32),
            pltpu.VMEM((N_ROW_CHUNKS, 128), jnp.int32),
            pltpu.VMEM((N_ROW_CHUNKS, 128), jnp.float32),
            pltpu.VMEM((N_ROW_CHUNKS, 128), jnp.float32),
            pltpu.VMEM((N_ROW_CHUNKS, 128), jnp.int32),
            pltpu.VMEM((ZCHUNK,), jnp.float32),
            pltpu.VMEM_SHARED((TBL_PAD,), jnp.float32),
            pltpu.VMEM_SHARED((TBL_PAD,), jnp.float32),
            pltpu.VMEM_SHARED((TBL_PAD,), jnp.float32),
            pltpu.SemaphoreType.DMA,
        ],
    )
    def _winner(h2d_hbm, out_hbm, h_v, idx_s, val_s, gat, w_v, zbuf,
                t0, t1, t2, sem):
        core = lax.axis_index("c")
        tile = lax.axis_index("s")
        tables = [t0, t1, t2]
        row0 = tile * ROWS_PER_TILE

        pltpu.sync_copy(h2d_hbm.at[pl.ds(tile * N_ROW_CHUNKS, N_ROW_CHUNKS)],
                        h_v)

        def _zb(k, _):
            zbuf[pl.ds(k * 16, 16)] = jnp.zeros((16,), jnp.float32)
            return 0
        lax.fori_loop(0, ZCHUNK // 16, _zb, 0)
        zofs = tile * (TBL_PAD // 16)
        for tb in tables:
            pltpu.sync_copy(zbuf.at[pl.ds(0, TBL_PAD // 16)],
                            tb.at[pl.ds(zofs, TBL_PAD // 16)])
        plsc.subcore_barrier()

        lane = lax.iota(jnp.int32, 16)
        for p, (spacing, gshift, gmask, gbits, wshift) in enumerate(_PASSES):
            def _mk(q, _):
                r = q >> 3
                o = (q & 7) * 16
                hh = h_v[r, pl.ds(o, 16)]
                ii = row0 + q * 16 + lane
                g = (ii >> gshift) & gmask
                val = _pow2_of_group(g, spacing, gbits)
                if p == 0:
                    idx = hh
                else:
                    w = w_v[r, pl.ds(o, 16)]
                    if p == 1:
                        act = (ii >> 10) == (w >> 10)
                    else:
                        act = (ii >> 5) == (w >> 5)
                    idx = jnp.where(act, hh, jnp.full((16,), DUMP, jnp.int32))
                idx_s[r, pl.ds(o, 16)] = idx
                val_s[r, pl.ds(o, 16)] = val
                return 0
            lax.fori_loop(0, N_ROW_CHUNKS * 8, _mk, 0)
            cps = [pltpu.async_copy(val_s.at[r], tables[p].at[idx_s.at[r]],
                                    sem, add=True)
                   for r in range(N_ROW_CHUNKS)]
            for cp in cps:
                cp.wait()
            plsc.subcore_barrier()
            cps = [pltpu.async_copy(tables[p].at[h_v.at[r]], gat.at[r], sem)
                   for r in range(N_ROW_CHUNKS)]
            for cp in cps:
                cp.wait()
            def _dec(q, _):
                r = q >> 3
                o = (q & 7) * 16
                s = gat[r, pl.ds(o, 16)]
                g = _top_group(s, spacing, gbits)
                if p == 0:
                    w_v[r, pl.ds(o, 16)] = g << wshift
                else:
                    w_v[r, pl.ds(o, 16)] = w_v[r, pl.ds(o, 16)] | (g << wshift)
                return 0
            lax.fori_loop(0, N_ROW_CHUNKS * 8, _dec, 0)

        half = N_ROW_CHUNKS // 2
        pltpu.sync_copy(
            w_v.at[pl.ds(core * half, half)],
            out_hbm.at[pl.ds(tile * N_ROW_CHUNKS + core * half, half)])

    return _winner


def _gather_build(nc, ns):
    nw = nc * ns
    b_per_w = BATCH // nw
    n_chunks = b_per_w // 128
    mesh = plsc.VectorSubcoreMesh(core_axis_name="c", subcore_axis_name="s")

    @functools.partial(
        pl.kernel, mesh=mesh,
        out_type=jax.ShapeDtypeStruct((BATCH, EMBED_PAD), jnp.float32),
        scratch_types=[
            pltpu.VMEM((n_chunks, 128), jnp.int32),
            pltpu.VMEM((2, 128, EMBED_PAD), jnp.float32),
            pltpu.SemaphoreType.DMA,
            pltpu.SemaphoreType.DMA,
        ],
    )
    def _gather(enc_hbm, widx_hbm, out_hbm, idx_v, rows_v, gsem, osem):
        wid = lax.axis_index("s") * nc + lax.axis_index("c")
        base = wid * b_per_w
        pltpu.sync_copy(widx_hbm.at[pl.ds(wid * n_chunks, n_chunks)], idx_v)
        cps = [None, None]
        ops = [None, None]
        cps[0] = pltpu.async_copy(enc_hbm.at[idx_v.at[0]], rows_v.at[0], gsem)
        for c in range(n_chunks):
            b = c % 2
            nb = (c + 1) % 2
            if c + 1 < n_chunks:
                if ops[nb] is not None:
                    ops[nb].wait()
                    ops[nb] = None
                cps[nb] = pltpu.async_copy(
                    enc_hbm.at[idx_v.at[c + 1]], rows_v.at[nb], gsem)
            cps[b].wait()
            ops[b] = pltpu.async_copy(
                rows_v.at[b], out_hbm.at[pl.ds(base + c * 128, 128)], osem)
        for op in ops:
            if op is not None:
                op.wait()

    return _gather


def kernel(lonlat, slepian_proj, norm_slep_w, norm_slep_b, norm_sh_w, norm_sh_b, cache_mem):
    del cache_mem
    lonlatT = lonlat.T
    w_slep = norm_slep_w.reshape(SLEP_DIM, 1)
    b_slep = norm_slep_b.reshape(SLEP_DIM, 1)
    w_sh = norm_sh_w.reshape(SH_DIM, 1)
    b_sh = norm_sh_b.reshape(SH_DIM, 1)

    enc = pl.pallas_call(
        _enc_kernel,
        grid=(BATCH // ENC_BLK,),
        in_specs=[
            pl.BlockSpec((2, ENC_BLK), lambda i: (0, i)),
            pl.BlockSpec((L_SLEP * L_SLEP, SLEP_DIM), lambda i: (0, 0)),
            pl.BlockSpec((SLEP_DIM, 1), lambda i: (0, 0)),
            pl.BlockSpec((SLEP_DIM, 1), lambda i: (0, 0)),
            pl.BlockSpec((SH_DIM, 1), lambda i: (0, 0)),
            pl.BlockSpec((SH_DIM, 1), lambda i: (0, 0)),
        ],
        out_specs=pl.BlockSpec((ENC_BLK, EMBED_PAD), lambda i: (i, 0)),
        out_shape=jax.ShapeDtypeStruct((BATCH, EMBED_PAD), jnp.float32),
        scratch_shapes=[pltpu.VMEM((L_SLEP * L_SLEP, ENC_BLK), jnp.float32)],
    )(lonlatT, slepian_proj, w_slep, b_slep, w_sh, b_sh)

    hashes = pl.pallas_call(
        _hash_kernel,
        in_specs=[pl.BlockSpec((2, BATCH), lambda: (0, 0))],
        out_specs=pl.BlockSpec((1, BATCH), lambda: (0, 0)),
        out_shape=jax.ShapeDtypeStruct((1, BATCH), jnp.int32),
    )(lonlatT)
    h2d = hashes.reshape(BATCH // 128, 128)

    info = plsc.get_sparse_core_info()
    winner = _winner_sc_build(info.num_cores, info.num_subcores)
    widx = winner(h2d)
    gather = _gather_build(info.num_cores, info.num_subcores)
    return gather(enc, widx)[:, :EMBED_DIM]


# speedup vs baseline: 36.6364x; 1.4185x over previous
import functools
import math

import jax
import jax.numpy as jnp
from jax import lax
from jax.experimental import pallas as pl
from jax.experimental.pallas import tpu as pltpu
from jax.experimental.pallas import tpu_sc as plsc

L_SLEP = 20
L_SH = 10
CACHE_SIZE = 200000
SLEP_DIM = 100
SH_DIM = L_SH * L_SH
EMBED_DIM = SLEP_DIM + SH_DIM
EMBED_PAD = 256
BATCH = 16384

ENC_BLK = 512
WIN_I = 2048
WIN_J = 512


def _feat_index(l, m):
    return l * l + (m + l)


def _enc_kernel(lonlatT_ref, proj_ref, wslep_ref, bslep_ref, wsh_ref, bsh_ref,
                eye_ref, out_ref, feat_ref):
    lon = jnp.deg2rad(lonlatT_ref[0:1, :])
    lat = jnp.deg2rad(lonlatT_ref[1:2, :])
    x = jnp.sin(lat)
    somx2 = jnp.sqrt(jnp.clip(1.0 - x * x, 0.0, 1.0))
    cos1 = jnp.cos(lon)
    sin1 = jnp.sin(lon)

    sqrt2 = math.sqrt(2.0)
    pmm = jnp.ones_like(x)
    cm, sm = jnp.ones_like(x), jnp.zeros_like(x)
    cm1, sm1 = None, None
    for m in range(L_SLEP):
        if m > 0:
            pmm = -(2 * m - 1) * somx2 * pmm
            if m == 1:
                cm1, sm1 = cm, sm
                cm, sm = cos1, sin1
            else:
                c_new = 2.0 * cos1 * cm - cm1
                s_new = 2.0 * cos1 * sm - sm1
                cm1, sm1 = cm, sm
                cm, sm = c_new, s_new
        pl2, pl1 = None, pmm
        for l in range(m, L_SLEP):
            if l == m:
                p = pmm
            elif l == m + 1:
                p = x * (2 * m + 1) * pmm
            else:
                p = ((2 * l - 1) * x * pl1 - (l + m - 1) * pl2) / (l - m)
            if l > m:
                pl2, pl1 = pl1, p
            n = math.sqrt((2 * l + 1) / (4 * math.pi)
                          * math.factorial(l - m) / math.factorial(l + m))
            base = n * p
            if m == 0:
                feat_ref[_feat_index(l, 0):_feat_index(l, 0) + 1, :] = base
            else:
                feat_ref[_feat_index(l, m):_feat_index(l, m) + 1, :] = sqrt2 * base * cm
                feat_ref[_feat_index(l, -m):_feat_index(l, -m) + 1, :] = sqrt2 * base * sm
    featT = feat_ref[...]
    slepT = lax.dot_general(proj_ref[...], featT,
                            (((0,), (0,)), ((), ())),
                            preferred_element_type=jnp.float32,
                            precision=lax.Precision.DEFAULT)
    shT = featT[0:SH_DIM, :]

    def _ln_T(v, w_col, b_col):
        mu = jnp.mean(v, axis=0, keepdims=True)
        var = jnp.mean((v - mu) * (v - mu), axis=0, keepdims=True)
        return (v - mu) / jnp.sqrt(var + 1e-5) * w_col + b_col

    slepT = _ln_T(slepT, wslep_ref[...], bslep_ref[...])
    shT = _ln_T(shT, wsh_ref[...], bsh_ref[...])
    encT = jnp.concatenate([slepT, shT], axis=0)

    r = out_ref.shape[0]
    out_ref[:, 0:EMBED_DIM] = lax.dot_general(
        eye_ref[...], encT, (((1,), (1,)), ((), ())),
        preferred_element_type=jnp.float32,
        precision=lax.Precision.DEFAULT)
    out_ref[:, EMBED_DIM:EMBED_PAD] = jnp.zeros(
        (r, EMBED_PAD - EMBED_DIM), jnp.float32)


def _hash_rows(q0, q1):
    h = q0 * jnp.int32(73856093) ^ q1 * jnp.int32(19349663)
    return jnp.mod(h, jnp.int32(CACHE_SIZE))


def _hash_kernel(lonlatT_ref, out_ref):
    qT = jnp.round(lonlatT_ref[...] * 1e4).astype(jnp.int32)
    out_ref[...] = _hash_rows(qT[0:1, :], qT[1:2, :])


TBL_PAD = 200192
DUMP = 200032
ROWS_PER_TILE = BATCH // 16
N_ROW_CHUNKS = ROWS_PER_TILE // 128
ZCHUNK = 3 * TBL_PAD // 16

_PASSES = (
    (11, 10, 0xF, 4, 10),
    (6, 5, 0x1F, 5, 5),
    (2, 0, 0x1F, 5, 0),
)


def _pow2_of_group(g, spacing, nbits):
    v = jnp.full((16,), 2.0 ** -126, jnp.float32)
    for b in range(nbits):
        f = jnp.float32(2.0 ** (spacing * (1 << b)))
        v = jnp.where((g & (1 << b)) != 0, v * f, v)
    return v


def _top_group(s, spacing, nbits):
    cur = jnp.full((16,), 2.0 ** -126, jnp.float32)
    g = jnp.zeros((16,), jnp.int32)
    for b in range(nbits - 1, -1, -1):
        trial = cur * jnp.float32(2.0 ** (spacing * (1 << b)))
        take = s >= trial
        cur = jnp.where(take, trial, cur)
        g = g | jnp.where(take, jnp.int32(1 << b), jnp.int32(0))
    return g


def _winner_sc_build(nc, ns):
    mesh = plsc.VectorSubcoreMesh(core_axis_name="c", subcore_axis_name="s")

    @functools.partial(
        pl.kernel, mesh=mesh,
        out_type=jax.ShapeDtypeStruct((BATCH // 128, 128), jnp.int32),
        scratch_types=[
            pltpu.VMEM((N_ROW_CHUNKS, 128), jnp.int32),
            pltpu.VMEM((N_ROW_CHUNKS, 128), jnp.int32),
            pltpu.VMEM((N_ROW_CHUNKS, 128), jnp.float32),
            pltpu.VMEM((N_ROW_CHUNKS, 128), jnp.float32),
            pltpu.VMEM((N_ROW_CHUNKS, 128), jnp.int32),
            pltpu.VMEM((TBL_PAD // 16,), jnp.float32),
            pltpu.VMEM_SHARED((TBL_PAD,), jnp.float32),
            pltpu.VMEM_SHARED((TBL_PAD,), jnp.float32),
            pltpu.VMEM_SHARED((TBL_PAD,), jnp.float32),
            pltpu.SemaphoreType.DMA,
        ],
    )
    def _winner(h2d_hbm, zeros_hbm, out_hbm, h_v, idx_s, val_s, gat, w_v,
                zbuf, t0, t1, t2, sem):
        core = lax.axis_index("c")
        tile = lax.axis_index("s")
        tables = [t0, t1, t2]
        row0 = tile * ROWS_PER_TILE

        pltpu.sync_copy(h2d_hbm.at[pl.ds(tile * N_ROW_CHUNKS, N_ROW_CHUNKS)],
                        h_v)

        zofs = tile * (TBL_PAD // 16)
        pltpu.sync_copy(zeros_hbm, zbuf)
        for tb in tables:
            pltpu.sync_copy(zbuf, tb.at[pl.ds(zofs, TBL_PAD // 16)])
        plsc.subcore_barrier()

        lane = lax.iota(jnp.int32, 16)
        for p, (spacing, gshift, gmask, gbits, wshift) in enumerate(_PASSES):
            def _mk(q, _):
                r = q >> 3
                o = (q & 7) * 16
                hh = h_v[r, pl.ds(o, 16)]
                ii = row0 + q * 16 + lane
                g = (ii >> gshift) & gmask
                val = _pow2_of_group(g, spacing, gbits)
                if p == 0:
                    idx = hh
                else:
                    w = w_v[r, pl.ds(o, 16)]
                    if p == 1:
                        act = (ii >> 10) == (w >> 10)
                    else:
                        act = (ii >> 5) == (w >> 5)
                    idx = jnp.where(act, hh, jnp.full((16,), DUMP, jnp.int32))
                idx_s[r, pl.ds(o, 16)] = idx
                val_s[r, pl.ds(o, 16)] = val
                return 0
            lax.fori_loop(0, N_ROW_CHUNKS * 8, _mk, 0)
            cps = [pltpu.async_copy(val_s.at[r], tables[p].at[idx_s.at[r]],
                                    sem, add=True)
                   for r in range(N_ROW_CHUNKS)]
            for cp in cps:
                cp.wait()
            plsc.subcore_barrier()
            cps = [pltpu.async_copy(tables[p].at[h_v.at[r]], gat.at[r], sem)
                   for r in range(N_ROW_CHUNKS)]
            for cp in cps:
                cp.wait()
            def _dec(q, _):
                r = q >> 3
                o = (q & 7) * 16
                s = gat[r, pl.ds(o, 16)]
                g = _top_group(s, spacing, gbits)
                if p == 0:
                    w_v[r, pl.ds(o, 16)] = g << wshift
                else:
                    w_v[r, pl.ds(o, 16)] = w_v[r, pl.ds(o, 16)] | (g << wshift)
                return 0
            lax.fori_loop(0, N_ROW_CHUNKS * 8, _dec, 0)

        half = N_ROW_CHUNKS // 2
        pltpu.sync_copy(
            w_v.at[pl.ds(core * half, half)],
            out_hbm.at[pl.ds(tile * N_ROW_CHUNKS + core * half, half)])

    return _winner


def _gather_build(nc, ns):
    nw = nc * ns
    b_per_w = BATCH // nw
    n_chunks = b_per_w // 128
    mesh = plsc.VectorSubcoreMesh(core_axis_name="c", subcore_axis_name="s")

    @functools.partial(
        pl.kernel, mesh=mesh,
        out_type=jax.ShapeDtypeStruct((BATCH, EMBED_PAD), jnp.float32),
        scratch_types=[
            pltpu.VMEM((n_chunks, 128), jnp.int32),
            pltpu.VMEM((2, 128, EMBED_PAD), jnp.float32),
            pltpu.SemaphoreType.DMA,
            pltpu.SemaphoreType.DMA,
        ],
    )
    def _gather(enc_hbm, widx_hbm, out_hbm, idx_v, rows_v, gsem, osem):
        wid = lax.axis_index("s") * nc + lax.axis_index("c")
        base = wid * b_per_w
        pltpu.sync_copy(widx_hbm.at[pl.ds(wid * n_chunks, n_chunks)], idx_v)
        cps = [None, None]
        ops = [None, None]
        cps[0] = pltpu.async_copy(enc_hbm.at[idx_v.at[0]], rows_v.at[0], gsem)
        for c in range(n_chunks):
            b = c % 2
            nb = (c + 1) % 2
            if c + 1 < n_chunks:
                if ops[nb] is not None:
                    ops[nb].wait()
                    ops[nb] = None
                cps[nb] = pltpu.async_copy(
                    enc_hbm.at[idx_v.at[c + 1]], rows_v.at[nb], gsem)
            cps[b].wait()
            ops[b] = pltpu.async_copy(
                rows_v.at[b], out_hbm.at[pl.ds(base + c * 128, 128)], osem)
        for op in ops:
            if op is not None:
                op.wait()

    return _gather


def kernel(lonlat, slepian_proj, norm_slep_w, norm_slep_b, norm_sh_w, norm_sh_b, cache_mem):
    del cache_mem
    lonlatT = lonlat.T
    w_slep = norm_slep_w.reshape(SLEP_DIM, 1)
    b_slep = norm_slep_b.reshape(SLEP_DIM, 1)
    w_sh = norm_sh_w.reshape(SH_DIM, 1)
    b_sh = norm_sh_b.reshape(SH_DIM, 1)

    enc = pl.pallas_call(
        _enc_kernel,
        grid=(BATCH // ENC_BLK,),
        in_specs=[
            pl.BlockSpec((2, ENC_BLK), lambda i: (0, i)),
            pl.BlockSpec((L_SLEP * L_SLEP, SLEP_DIM), lambda i: (0, 0)),
            pl.BlockSpec((SLEP_DIM, 1), lambda i: (0, 0)),
            pl.BlockSpec((SLEP_DIM, 1), lambda i: (0, 0)),
            pl.BlockSpec((SH_DIM, 1), lambda i: (0, 0)),
            pl.BlockSpec((SH_DIM, 1), lambda i: (0, 0)),
            pl.BlockSpec((ENC_BLK, ENC_BLK), lambda i: (0, 0)),
        ],
        out_specs=pl.BlockSpec((ENC_BLK, EMBED_PAD), lambda i: (i, 0)),
        out_shape=jax.ShapeDtypeStruct((BATCH, EMBED_PAD), jnp.float32),
        scratch_shapes=[pltpu.VMEM((L_SLEP * L_SLEP, ENC_BLK), jnp.float32)],
    )(lonlatT, slepian_proj, w_slep, b_slep, w_sh, b_sh,
      jnp.eye(ENC_BLK, dtype=jnp.float32))

    hashes = pl.pallas_call(
        _hash_kernel,
        in_specs=[pl.BlockSpec((2, BATCH), lambda: (0, 0))],
        out_specs=pl.BlockSpec((1, BATCH), lambda: (0, 0)),
        out_shape=jax.ShapeDtypeStruct((1, BATCH), jnp.int32),
    )(lonlatT)
    h2d = hashes.reshape(BATCH // 128, 128)

    info = plsc.get_sparse_core_info()
    winner = _winner_sc_build(info.num_cores, info.num_subcores)
    widx = winner(h2d, jnp.zeros((TBL_PAD // 16,), jnp.float32))
    gather = _gather_build(info.num_cores, info.num_subcores)
    return gather(enc, widx)[:, :EMBED_DIM]
